# Initial kernel scaffold; baseline (speedup 1.0000x reference)
import jax
import jax.numpy as jnp
from jax.experimental import pallas as pl


def kernel(x, edge_index, W_self1, W_neigh1, b1, W_self2, W_neigh2, b2, W_self3, W_neigh3, b3):
    raise NotImplementedError("write your pallas kernel here")


# speedup vs baseline: 3.2987x; 3.2987x over previous
import jax
import jax.numpy as jnp
from jax import lax
from jax.experimental import pallas as pl
from jax.experimental.pallas import tpu as pltpu
from jax.experimental.pallas import tpu_sc as plsc

N = 10000
E = 320000
NSUB = 16
NCORE = 2
WIN = 128
NWIN_ES = 80
NWIN_CS = 160
CHUNK = 16
N_PAD = 10112
RPT = N_PAD // NSUB
TRASH = N


def _make_segsum(nwin, with_deg):
    mesh = plsc.VectorSubcoreMesh(core_axis_name="c", subcore_axis_name="s",
                                  num_cores=NCORE, num_subcores=NSUB)
    out_type = [jax.ShapeDtypeStruct((NCORE, N_PAD, 128), jnp.float32)]
    scratch = [
        pltpu.VMEM_SHARED((N_PAD, 128), jnp.float32),
        pltpu.VMEM((CHUNK, WIN), jnp.int32),
        pltpu.VMEM((CHUNK, WIN), jnp.int32),
        pltpu.VMEM((WIN, 128), jnp.float32),
    ]
    if with_deg:
        out_type.append(jax.ShapeDtypeStruct((NCORE, 1, N_PAD), jnp.float32))
        scratch += [
            pltpu.VMEM_SHARED((N_PAD,), jnp.float32),
            pltpu.VMEM((WIN,), jnp.float32),
        ]

    def inner(table, idxs, dsts, zeros, out, acc, gbuf, dbuf, rows, deg):
        c = lax.axis_index("c")
        s = lax.axis_index("s")
        r0 = s * RPT
        pltpu.sync_copy(zeros.at[pl.ds(r0, RPT)], acc.at[pl.ds(r0, RPT)])
        if deg is not None:
            zeros1, ones1, degout, dacc, ones_v = deg
            pltpu.sync_copy(ones1, ones_v)

            @pl.when(s == 0)
            def _():
                pltpu.sync_copy(zeros1, dacc)
        plsc.subcore_barrier()

        def chunk_body(k, carry):
            pltpu.sync_copy(idxs.at[c, s, pl.ds(k * CHUNK, CHUNK)], gbuf)
            pltpu.sync_copy(dsts.at[c, s, pl.ds(k * CHUNK, CHUNK)], dbuf)

            def win(j, carry2):
                pltpu.sync_copy(table.at[gbuf.at[j]], rows)
                pltpu.sync_copy(rows, acc.at[dbuf.at[j]], add=True)
                if deg is not None:
                    pltpu.sync_copy(deg[4], deg[3].at[dbuf.at[j]], add=True)
                return carry2

            return lax.fori_loop(0, CHUNK, win, carry)

        lax.fori_loop(0, nwin // CHUNK, chunk_body, 0)
        plsc.subcore_barrier()
        pltpu.sync_copy(acc.at[pl.ds(r0, RPT)], out.at[c, pl.ds(r0, RPT)])
        if deg is not None:
            @pl.when(s == 0)
            def _():
                pltpu.sync_copy(deg[3], deg[2].at[c, 0])

    if with_deg:
        def body(table, idxs, dsts, zeros, zeros1, ones1, out, degout,
                 acc, gbuf, dbuf, rows, dacc, ones_v):
            inner(table, idxs, dsts, zeros, out, acc, gbuf, dbuf, rows,
                  (zeros1, ones1, degout, dacc, ones_v))
    else:
        def body(table, idxs, dsts, zeros, out, acc, gbuf, dbuf, rows):
            inner(table, idxs, dsts, zeros, out, acc, gbuf, dbuf, rows, None)

    return pl.kernel(body, out_type=tuple(out_type), mesh=mesh,
                     scratch_types=tuple(scratch))


_segsum_es_deg = _make_segsum(NWIN_ES, True)
_segsum_cs = _make_segsum(NWIN_CS, False)
_segsum_es = _make_segsum(NWIN_ES, False)


_BR = 1000


def _dense12(Din, split_cols, extra):
    Dout = 256
    in_specs = [
        pl.BlockSpec((_BR, Din), lambda i: (i, 0)),
        pl.BlockSpec((_BR, 128), lambda i: (i, 0)),
        pl.BlockSpec((_BR, 128), lambda i: (i, 0)),
        pl.BlockSpec((_BR, 1), lambda i: (i, 0)),
        pl.BlockSpec((_BR, 1), lambda i: (i, 0)),
        pl.BlockSpec((Din, Dout), lambda i: (0, 0)),
        pl.BlockSpec((128, Dout), lambda i: (0, 0)),
        pl.BlockSpec((128, Dout), lambda i: (0, 0)),
        pl.BlockSpec((Dout,), lambda i: (0,)),
    ]
    out_shape = [jax.ShapeDtypeStruct((N, Dout), jnp.float32)]
    out_specs = [pl.BlockSpec((_BR, Dout), lambda i: (i, 0))]
    if extra:
        in_specs.append(pl.BlockSpec((Dout, 128), lambda i: (0, 0)))
        out_shape.append(jax.ShapeDtypeStruct((N, 128), jnp.float32))
        out_specs.append(pl.BlockSpec((_BR, 128), lambda i: (i, 0)))

    def body(h, s0, s1, dga, dgb, ws, wnt, wnb, b, *rest):
        if extra:
            wx, o_ref, t_ref = rest
        else:
            (o_ref,) = rest
        inv = 1.0 / jnp.maximum(dga[...] + dgb[...], 1.0)
        if split_cols:
            neigh = jnp.dot(s0[...], wnt[...],
                            preferred_element_type=jnp.float32)
            neigh = neigh + jnp.dot(s1[...], wnb[...],
                                    preferred_element_type=jnp.float32)
        else:
            neigh = jnp.dot(s0[...] + s1[...], wnt[...],
                            preferred_element_type=jnp.float32)
        o = jnp.dot(h[...], ws[...], preferred_element_type=jnp.float32)
        o = o + neigh * inv + b[...][None, :]
        o = jnp.maximum(o, 0.0)
        o_ref[...] = o
        if extra:
            t_ref[...] = jnp.dot(o, wx[...], preferred_element_type=jnp.float32)

    return pl.pallas_call(body, grid=(N // _BR,), in_specs=in_specs,
                          out_specs=out_specs, out_shape=out_shape)


def _dense3():
    Din, Dout = 256, 128
    in_specs = [
        pl.BlockSpec((_BR, Din), lambda i: (i, 0)),
        pl.BlockSpec((_BR, 128), lambda i: (i, 0)),
        pl.BlockSpec((_BR, 128), lambda i: (i, 0)),
        pl.BlockSpec((_BR, 1), lambda i: (i, 0)),
        pl.BlockSpec((_BR, 1), lambda i: (i, 0)),
        pl.BlockSpec((Din, Dout), lambda i: (0, 0)),
        pl.BlockSpec((Dout,), lambda i: (0,)),
    ]

    def body(h, s0, s1, dga, dgb, ws, b, o_ref):
        inv = 1.0 / jnp.maximum(dga[...] + dgb[...], 1.0)
        neigh = s0[...] + s1[...]
        o = jnp.dot(h[...], ws[...], preferred_element_type=jnp.float32)
        o_ref[...] = o + neigh * inv + b[...][None, :]

    return pl.pallas_call(
        body, grid=(N // _BR,), in_specs=in_specs,
        out_specs=pl.BlockSpec((_BR, Dout), lambda i: (i, 0)),
        out_shape=jax.ShapeDtypeStruct((N, Dout), jnp.float32))


_tc1 = _dense12(128, False, False)
_tc2 = _dense12(256, True, True)
_tc3 = _dense3()


def _pad_reshape(a, per_tile_pad, fill, nwin):
    half = a.shape[1] // NSUB
    a = a.reshape(NCORE, NSUB, half)
    padn = nwin * WIN - half
    a = jnp.pad(a, ((0, 0), (0, 0), (0, padn)), constant_values=fill)
    return a.reshape(NCORE, NSUB, nwin, WIN)


def kernel(x, edge_index, W_self1, W_neigh1, b1, W_self2, W_neigh2, b2,
           W_self3, W_neigh3, b3):
    src = edge_index[0].astype(jnp.int32)
    dst = edge_index[1].astype(jnp.int32)

    src_es = _pad_reshape(src.reshape(NCORE, E // NCORE), None, 0, NWIN_ES)
    dst_es = _pad_reshape(dst.reshape(NCORE, E // NCORE), None, TRASH,
                          NWIN_ES)
    idx_cs = jnp.stack([src * 2, src * 2 + 1]).reshape(NCORE, NSUB, -1)
    padn = NWIN_CS * WIN - idx_cs.shape[2]
    idx_cs = jnp.pad(idx_cs, ((0, 0), (0, 0), (0, padn))).reshape(
        NCORE, NSUB, NWIN_CS, WIN)
    dst_cs = jnp.pad(jnp.stack([dst, dst]).reshape(NCORE, NSUB, -1),
                     ((0, 0), (0, 0), (0, padn)),
                     constant_values=TRASH).reshape(NCORE, NSUB, NWIN_CS, WIN)

    zeros = jnp.zeros((N_PAD, 128), jnp.float32)
    zeros1 = jnp.zeros((N_PAD,), jnp.float32)
    ones1 = jnp.ones((WIN,), jnp.float32)

    s1p, degp = _segsum_es_deg(x, src_es, dst_es, zeros, zeros1, ones1)
    dga, dgb = degp[0, 0, :N, None], degp[1, 0, :N, None]
    h1 = _tc1(x, s1p[0, :N], s1p[1, :N], dga, dgb,
              W_self1, W_neigh1, W_neigh1, b1)[0]
    (s2p,) = _segsum_cs(h1.reshape(2 * N, 128), idx_cs, dst_cs, zeros)
    h2, t3 = _tc2(h1, s2p[0, :N], s2p[1, :N], dga, dgb,
                  W_self2, W_neigh2[:128], W_neigh2[128:], b2, W_neigh3)
    (s3p,) = _segsum_es(t3, src_es, dst_es, zeros)
    out = _tc3(h2, s3p[0, :N], s3p[1, :N], dga, dgb, W_self3, b3)
    return out


# speedup vs baseline: 3.7810x; 1.1462x over previous
import jax
import jax.numpy as jnp
from jax import lax
from jax.experimental import pallas as pl
from jax.experimental.pallas import tpu as pltpu
from jax.experimental.pallas import tpu_sc as plsc

N = 10000
E = 320000
NSUB = 16
NCORE = 2
WIN = 128
NWIN_ES = 80
NWIN_CS = 160
CHUNK = 16
N_PAD = 10112
RPT = N_PAD // NSUB
TRASH = N


def _make_segsum(nwin, with_deg):
    mesh = plsc.VectorSubcoreMesh(core_axis_name="c", subcore_axis_name="s",
                                  num_cores=NCORE, num_subcores=NSUB)
    out_type = [jax.ShapeDtypeStruct((NCORE, N_PAD, 128), jnp.float32)]
    scratch = [
        pltpu.VMEM_SHARED((N_PAD, 128), jnp.float32),
        pltpu.VMEM((CHUNK, WIN), jnp.int32),
        pltpu.VMEM((CHUNK, WIN), jnp.int32),
        pltpu.VMEM((2, WIN, 128), jnp.float32),
        pltpu.SemaphoreType.DMA,
        pltpu.SemaphoreType.DMA,
    ]
    if with_deg:
        out_type.append(jax.ShapeDtypeStruct((NCORE, 1, N_PAD), jnp.float32))
        scratch += [
            pltpu.VMEM_SHARED((N_PAD,), jnp.float32),
            pltpu.VMEM((WIN,), jnp.float32),
        ]

    def inner(table, idxs, dsts, zeros, out, acc, gbuf, dbuf, rows, gsem,
              ssem, deg):
        c = lax.axis_index("c")
        s = lax.axis_index("s")
        r0 = s * RPT
        pltpu.sync_copy(zeros.at[pl.ds(r0, RPT)], acc.at[pl.ds(r0, RPT)])
        if deg is not None:
            zeros1, ones1, degout, dacc, ones_v = deg
            pltpu.sync_copy(ones1, ones_v)

            @pl.when(s == 0)
            def _():
                pltpu.sync_copy(zeros1, dacc)
        plsc.subcore_barrier()

        def wait_rows(sem, buf):
            pltpu.make_async_copy(table.at[pl.ds(0, WIN)], rows.at[buf],
                                  sem).wait()

        def chunk_body(k, carry):
            pltpu.sync_copy(idxs.at[c, s, pl.ds(k * CHUNK, CHUNK)], gbuf)
            pltpu.sync_copy(dsts.at[c, s, pl.ds(k * CHUNK, CHUNK)], dbuf)
            pltpu.async_copy(table.at[gbuf.at[0]], rows.at[0], gsem)
            for j in range(CHUNK):
                if j + 1 < CHUNK:
                    if j >= 1:
                        wait_rows(ssem, (j + 1) % 2)
                    pltpu.async_copy(table.at[gbuf.at[j + 1]],
                                     rows.at[(j + 1) % 2], gsem)
                wait_rows(gsem, j % 2)
                pltpu.async_copy(rows.at[j % 2], acc.at[dbuf.at[j]], ssem,
                                 add=True)
                if deg is not None:
                    pltpu.sync_copy(deg[4], deg[3].at[dbuf.at[j]], add=True)
            wait_rows(ssem, 0)
            wait_rows(ssem, 1)
            return carry

        lax.fori_loop(0, nwin // CHUNK, chunk_body, 0)
        plsc.subcore_barrier()
        pltpu.sync_copy(acc.at[pl.ds(r0, RPT)], out.at[c, pl.ds(r0, RPT)])
        if deg is not None:
            @pl.when(s == 0)
            def _():
                pltpu.sync_copy(deg[3], deg[2].at[c, 0])

    if with_deg:
        def body(table, idxs, dsts, zeros, zeros1, ones1, out, degout,
                 acc, gbuf, dbuf, rows, gsem, ssem, dacc, ones_v):
            inner(table, idxs, dsts, zeros, out, acc, gbuf, dbuf, rows,
                  gsem, ssem, (zeros1, ones1, degout, dacc, ones_v))
    else:
        def body(table, idxs, dsts, zeros, out, acc, gbuf, dbuf, rows,
                 gsem, ssem):
            inner(table, idxs, dsts, zeros, out, acc, gbuf, dbuf, rows,
                  gsem, ssem, None)

    return pl.kernel(body, out_type=tuple(out_type), mesh=mesh,
                     scratch_types=tuple(scratch))


_segsum_es_deg = _make_segsum(NWIN_ES, True)
_segsum_cs = _make_segsum(NWIN_CS, False)
_segsum_es = _make_segsum(NWIN_ES, False)


_BR = 1000


def _dense12(Din, split_cols, extra):
    Dout = 256
    in_specs = [
        pl.BlockSpec((_BR, Din), lambda i: (i, 0)),
        pl.BlockSpec((_BR, 128), lambda i: (i, 0)),
        pl.BlockSpec((_BR, 128), lambda i: (i, 0)),
        pl.BlockSpec((_BR, 1), lambda i: (i, 0)),
        pl.BlockSpec((_BR, 1), lambda i: (i, 0)),
        pl.BlockSpec((Din, Dout), lambda i: (0, 0)),
        pl.BlockSpec((128, Dout), lambda i: (0, 0)),
        pl.BlockSpec((128, Dout), lambda i: (0, 0)),
        pl.BlockSpec((Dout,), lambda i: (0,)),
    ]
    out_shape = [jax.ShapeDtypeStruct((N, Dout), jnp.float32)]
    out_specs = [pl.BlockSpec((_BR, Dout), lambda i: (i, 0))]
    if extra:
        in_specs.append(pl.BlockSpec((Dout, 128), lambda i: (0, 0)))
        out_shape.append(jax.ShapeDtypeStruct((N, 128), jnp.float32))
        out_specs.append(pl.BlockSpec((_BR, 128), lambda i: (i, 0)))

    def body(h, s0, s1, dga, dgb, ws, wnt, wnb, b, *rest):
        if extra:
            wx, o_ref, t_ref = rest
        else:
            (o_ref,) = rest
        inv = 1.0 / jnp.maximum(dga[...] + dgb[...], 1.0)
        if split_cols:
            neigh = jnp.dot(s0[...], wnt[...],
                            preferred_element_type=jnp.float32)
            neigh = neigh + jnp.dot(s1[...], wnb[...],
                                    preferred_element_type=jnp.float32)
        else:
            neigh = jnp.dot(s0[...] + s1[...], wnt[...],
                            preferred_element_type=jnp.float32)
        o = jnp.dot(h[...], ws[...], preferred_element_type=jnp.float32)
        o = o + neigh * inv + b[...][None, :]
        o = jnp.maximum(o, 0.0)
        o_ref[...] = o
        if extra:
            t_ref[...] = jnp.dot(o, wx[...], preferred_element_type=jnp.float32)

    return pl.pallas_call(body, grid=(N // _BR,), in_specs=in_specs,
                          out_specs=out_specs, out_shape=out_shape)


def _dense3():
    Din, Dout = 256, 128
    in_specs = [
        pl.BlockSpec((_BR, Din), lambda i: (i, 0)),
        pl.BlockSpec((_BR, 128), lambda i: (i, 0)),
        pl.BlockSpec((_BR, 128), lambda i: (i, 0)),
        pl.BlockSpec((_BR, 1), lambda i: (i, 0)),
        pl.BlockSpec((_BR, 1), lambda i: (i, 0)),
        pl.BlockSpec((Din, Dout), lambda i: (0, 0)),
        pl.BlockSpec((Dout,), lambda i: (0,)),
    ]

    def body(h, s0, s1, dga, dgb, ws, b, o_ref):
        inv = 1.0 / jnp.maximum(dga[...] + dgb[...], 1.0)
        neigh = s0[...] + s1[...]
        o = jnp.dot(h[...], ws[...], preferred_element_type=jnp.float32)
        o_ref[...] = o + neigh * inv + b[...][None, :]

    return pl.pallas_call(
        body, grid=(N // _BR,), in_specs=in_specs,
        out_specs=pl.BlockSpec((_BR, Dout), lambda i: (i, 0)),
        out_shape=jax.ShapeDtypeStruct((N, Dout), jnp.float32))


_tc1 = _dense12(128, False, False)
_tc2 = _dense12(256, True, True)
_tc3 = _dense3()


def _pad_reshape(a, per_tile_pad, fill, nwin):
    half = a.shape[1] // NSUB
    a = a.reshape(NCORE, NSUB, half)
    padn = nwin * WIN - half
    a = jnp.pad(a, ((0, 0), (0, 0), (0, padn)), constant_values=fill)
    return a.reshape(NCORE, NSUB, nwin, WIN)


def kernel(x, edge_index, W_self1, W_neigh1, b1, W_self2, W_neigh2, b2,
           W_self3, W_neigh3, b3):
    src = edge_index[0].astype(jnp.int32)
    dst = edge_index[1].astype(jnp.int32)

    src_es = _pad_reshape(src.reshape(NCORE, E // NCORE), None, 0, NWIN_ES)
    dst_es = _pad_reshape(dst.reshape(NCORE, E // NCORE), None, TRASH,
                          NWIN_ES)
    idx_cs = jnp.stack([src * 2, src * 2 + 1]).reshape(NCORE, NSUB, -1)
    padn = NWIN_CS * WIN - idx_cs.shape[2]
    idx_cs = jnp.pad(idx_cs, ((0, 0), (0, 0), (0, padn))).reshape(
        NCORE, NSUB, NWIN_CS, WIN)
    dst_cs = jnp.pad(jnp.stack([dst, dst]).reshape(NCORE, NSUB, -1),
                     ((0, 0), (0, 0), (0, padn)),
                     constant_values=TRASH).reshape(NCORE, NSUB, NWIN_CS, WIN)

    zeros = jnp.zeros((N_PAD, 128), jnp.float32)
    zeros1 = jnp.zeros((N_PAD,), jnp.float32)
    ones1 = jnp.ones((WIN,), jnp.float32)

    s1p, degp = _segsum_es_deg(x, src_es, dst_es, zeros, zeros1, ones1)
    dga, dgb = degp[0, 0, :N, None], degp[1, 0, :N, None]
    h1 = _tc1(x, s1p[0, :N], s1p[1, :N], dga, dgb,
              W_self1, W_neigh1, W_neigh1, b1)[0]
    (s2p,) = _segsum_cs(h1.reshape(2 * N, 128), idx_cs, dst_cs, zeros)
    h2, t3 = _tc2(h1, s2p[0, :N], s2p[1, :N], dga, dgb,
                  W_self2, W_neigh2[:128], W_neigh2[128:], b2, W_neigh3)
    (s3p,) = _segsum_es(t3, src_es, dst_es, zeros)
    out = _tc3(h2, s3p[0, :N], s3p[1, :N], dga, dgb, W_self3, b3)
    return out
